# baseline (speedup 1.0000x reference)
import functools

import jax
import jax.numpy as jnp
from jax import lax
from jax.experimental import pallas as pl
from jax.experimental.pallas import tpu as pltpu
from jax.experimental.pallas import tpu_sc as plsc

NC = 2
NS = 16
L = 16


def _transform_body(x_ref, w_ref, o_ref):
    o_ref[...] = jnp.dot(x_ref[...], w_ref[0],
                         preferred_element_type=jnp.float32)


def _transform(xb, Wb, bn=2000):
    n, d = xb.shape
    r = Wb.shape[0]
    nb = n // bn
    return pl.pallas_call(
        _transform_body,
        grid=(nb, r),
        in_specs=[
            pl.BlockSpec((bn, d), lambda ni, ri: (ni, 0)),
            pl.BlockSpec((1, d, d), lambda ni, ri: (ri, 0, 0)),
        ],
        out_specs=pl.BlockSpec((bn, d), lambda ni, ri, _nb=nb: (ri * _nb + ni, 0)),
        out_shape=jax.ShapeDtypeStruct((r * n, d), jnp.float32),
    )(xb, Wb)


def _combine_body(p_ref, x_ref, w_ref, b_ref, o_ref):
    agg = p_ref[0] + p_ref[1]
    root = jnp.dot(x_ref[...], w_ref[...], preferred_element_type=jnp.float32)
    o_ref[...] = jnp.maximum(agg + root + b_ref[...], 0.0)


def _combine(partials, x, W_root, b2d, bn=1000):
    n, d = x.shape
    nb = n // bn
    return pl.pallas_call(
        _combine_body,
        grid=(nb,),
        in_specs=[
            pl.BlockSpec((NC, bn, d), lambda i: (0, i, 0)),
            pl.BlockSpec((bn, d), lambda i: (i, 0)),
            pl.BlockSpec((d, d), lambda i: (0, 0)),
            pl.BlockSpec((1, d), lambda i: (0, 0)),
        ],
        out_specs=pl.BlockSpec((bn, d), lambda i: (i, 0)),
        out_shape=jax.ShapeDtypeStruct((n, d), jnp.float32),
    )(partials, x, W_root, b2d)


def _sc_aggregate(xt, ei_flat, etype, enorm, n):
    e = etype.shape[0]
    d = xt.shape[1]
    nw = NC * NS
    epw = e // nw
    c = 80
    sb = 2000
    nsuper = epw // sb
    nch = sb // c
    nv_groups = c // L
    nrc = n // c
    kmax = (nrc + NS - 1) // NS
    mesh = plsc.VectorSubcoreMesh(core_axis_name="c", subcore_axis_name="s",
                                  num_cores=NC, num_subcores=NS)

    @functools.partial(
        pl.kernel,
        out_type=jax.ShapeDtypeStruct((NC, n, d), jnp.float32),
        mesh=mesh,
        scratch_types=[
            pltpu.VMEM((sb,), jnp.int32),
            pltpu.VMEM((sb,), jnp.int32),
            pltpu.VMEM((sb,), jnp.int32),
            pltpu.VMEM((sb,), jnp.float32),
            pltpu.VMEM((nch, c), jnp.int32),
            pltpu.VMEM((nch, c), jnp.int32),
            pltpu.VMEM((c, d), jnp.float32),
            pltpu.VMEM((c, d), jnp.float32),
            pltpu.VMEM((c, d), jnp.float32),
            pltpu.VMEM_SHARED((n, d), jnp.float32),
            pltpu.SemaphoreType.DMA,
            pltpu.SemaphoreType.DMA,
            pltpu.SemaphoreType.DMA,
            pltpu.SemaphoreType.DMA,
            pltpu.SemaphoreType.DMA,
            pltpu.SemaphoreType.DMA,
        ],
    )
    def agg_kernel(xt_hbm, ei_hbm, type_hbm, norm_hbm, out_hbm,
                   srcv, typev, dstv, normv, gidx2, didx2,
                   rows0, rows1, rows2, acc,
                   gsem0, gsem1, gsem2, ssem0, ssem1, ssem2):
        bufs = (rows0, rows1, rows2)
        gsems = (gsem0, gsem1, gsem2)
        ssems = (ssem0, ssem1, ssem2)
        cid = lax.axis_index("c")
        sid = lax.axis_index("s")
        wid = cid * NS + sid
        base = wid * epw

        zero = jnp.zeros((L,), jnp.float32)
        def zrows(i, carry):
            for j in range(d // L):
                rows0[i, pl.ds(j * L, L)] = zero
            return carry
        lax.fori_loop(0, c, zrows, 0)

        def zacc(k, carry):
            rc = sid + k * NS
            @pl.when(rc < nrc)
            def _():
                pltpu.sync_copy(rows0, acc.at[pl.ds(rc * c, c)])
            return carry
        lax.fori_loop(0, kmax, zacc, 0)
        plsc.subcore_barrier()

        dnums = lax.GatherDimensionNumbers(
            offset_dims=(), collapsed_slice_dims=(0,), start_index_map=(0,))

        def start_gather(i, b):
            @pl.when(i < nch)
            def _():
                pltpu.async_copy(xt_hbm.at[gidx2.at[i]], bufs[b], gsems[b])

        def wait_gather(b):
            pltpu.make_async_copy(
                xt_hbm.at[gidx2.at[0]], bufs[b], gsems[b]).wait()

        def start_scatter(i, b):
            pltpu.async_copy(bufs[b], acc.at[didx2.at[i]], ssems[b], add=True)

        def wait_scatter(b):
            pltpu.make_async_copy(
                bufs[b], acc.at[didx2.at[0]], ssems[b]).wait()

        def scale(i, b):
            buf = bufs[b]
            def group(g, inner2):
                nv = normv[pl.ds(i * c + g * L, L)]
                for jl in range(L):
                    lane = jnp.full((L, 1), jl, jnp.int32)
                    bc = lax.gather(
                        nv, lane, dnums, (1,),
                        mode=lax.GatherScatterMode.PROMISE_IN_BOUNDS)
                    row = g * L + jl
                    for k in range(d // L):
                        sl = pl.ds(k * L, L)
                        buf[row, sl] = buf[row, sl] * bc
                return inner2
            lax.fori_loop(0, nv_groups, group, 0)

        ntrip = (nch + 2) // 3

        def super_chunk(si, carry):
            sbase = base + si * sb
            pltpu.sync_copy(ei_hbm.at[pl.ds(sbase, sb)], srcv)
            pltpu.sync_copy(type_hbm.at[pl.ds(sbase, sb)], typev)
            pltpu.sync_copy(ei_hbm.at[pl.ds(e + sbase, sb)], dstv)
            pltpu.sync_copy(norm_hbm.at[pl.ds(sbase, sb)], normv)

            def prep(i, inner):
                for j in range(nv_groups):
                    off = i * c + j * L
                    s = srcv[pl.ds(off, L)]
                    t = typev[pl.ds(off, L)]
                    gidx2[i, pl.ds(j * L, L)] = t * n + s
                    didx2[i, pl.ds(j * L, L)] = dstv[pl.ds(off, L)]
                return inner
            lax.fori_loop(0, nch, prep, 0)

            start_gather(0, 0)
            start_gather(1, 1)

            def triple(j, inner):
                i = 3 * j
                wait_gather(0)
                scale(i, 0)
                start_scatter(i, 0)
                @pl.when(i > 0)
                def _():
                    wait_scatter(2)
                start_gather(i + 2, 2)
                @pl.when(i + 1 < nch)
                def _():
                    wait_gather(1)
                    scale(i + 1, 1)
                    start_scatter(i + 1, 1)
                @pl.when(i + 3 < nch)
                def _():
                    wait_scatter(0)
                    start_gather(i + 3, 0)
                @pl.when(i + 2 < nch)
                def _():
                    wait_gather(2)
                    scale(i + 2, 2)
                    start_scatter(i + 2, 2)
                @pl.when(i + 4 < nch)
                def _():
                    wait_scatter(1)
                    start_gather(i + 4, 1)
                return inner
            lax.fori_loop(0, ntrip, triple, 0)

            waited = set()
            for j in range(ntrip):
                if j > 0:
                    waited.add(3 * j - 1)
                if 3 * j + 3 < nch:
                    waited.add(3 * j)
                if 3 * j + 4 < nch and 3 * j + 1 < nch:
                    waited.add(3 * j + 1)
            for k in range(nch):
                if k not in waited:
                    wait_scatter(k % 3)
            return carry
        lax.fori_loop(0, nsuper, super_chunk, 0)
        plsc.subcore_barrier()

        def dump(k, carry):
            rc = sid + k * NS
            @pl.when(rc < nrc)
            def _():
                sl = pl.ds(rc * c, c)
                pltpu.sync_copy(acc.at[sl], rows0)
                pltpu.sync_copy(rows0, out_hbm.at[cid, sl])
            return carry
        lax.fori_loop(0, kmax, dump, 0)

    return agg_kernel(xt, ei_flat, etype, enorm)


def kernel(x, edge_index, edge_type, edge_norm, W_rel, W_root, b):
    n, d = x.shape
    xt = _transform(x.astype(jnp.bfloat16), W_rel.astype(jnp.bfloat16))
    partials = _sc_aggregate(xt, edge_index.reshape(-1), edge_type,
                             edge_norm, n)
    return _combine(partials, x, W_root, b.reshape(1, d))


# speedup vs baseline: 1.1312x; 1.1312x over previous
import functools

import jax
import jax.numpy as jnp
from jax import lax
from jax.experimental import pallas as pl
from jax.experimental.pallas import tpu as pltpu
from jax.experimental.pallas import tpu_sc as plsc

NC = 2
NS = 16
L = 16


def _transform_body(x_ref, w_ref, o_ref):
    o_ref[...] = jnp.dot(x_ref[...], w_ref[0],
                         preferred_element_type=jnp.float32)


def _transform(xb, Wb):
    n, d = xb.shape
    r = Wb.shape[0]
    return pl.pallas_call(
        _transform_body,
        grid=(r,),
        in_specs=[
            pl.BlockSpec((n, d), lambda ri: (0, 0)),
            pl.BlockSpec((1, d, d), lambda ri: (ri, 0, 0)),
        ],
        out_specs=pl.BlockSpec((n, d), lambda ri: (ri, 0)),
        out_shape=jax.ShapeDtypeStruct((r * n, d), jnp.float32),
    )(xb, Wb)


def _combine_body(p_ref, x_ref, w_ref, b_ref, o_ref):
    agg = p_ref[0] + p_ref[1]
    root = jnp.dot(x_ref[...], w_ref[...], preferred_element_type=jnp.float32)
    o_ref[...] = jnp.maximum(agg + root + b_ref[...], 0.0)


def _combine(partials, x, W_root, b2d, bn=2000):
    n, d = x.shape
    nb = n // bn
    return pl.pallas_call(
        _combine_body,
        grid=(nb,),
        in_specs=[
            pl.BlockSpec((NC, bn, d), lambda i: (0, i, 0)),
            pl.BlockSpec((bn, d), lambda i: (i, 0)),
            pl.BlockSpec((d, d), lambda i: (0, 0)),
            pl.BlockSpec((1, d), lambda i: (0, 0)),
        ],
        out_specs=pl.BlockSpec((bn, d), lambda i: (i, 0)),
        out_shape=jax.ShapeDtypeStruct((n, d), jnp.float32),
    )(partials, x, W_root, b2d)


def _sc_aggregate(xt, ei_flat, etype, enorm, n):
    e = etype.shape[0]
    d = xt.shape[1]
    nw = NC * NS
    epw = e // nw
    c = 80
    sb = 2000
    nsuper = epw // sb
    nch = sb // c
    nv_groups = c // L
    nrc = n // c
    kmax = (nrc + NS - 1) // NS
    mesh = plsc.VectorSubcoreMesh(core_axis_name="c", subcore_axis_name="s",
                                  num_cores=NC, num_subcores=NS)

    @functools.partial(
        pl.kernel,
        out_type=jax.ShapeDtypeStruct((NC, n, d), jnp.float32),
        mesh=mesh,
        scratch_types=[
            pltpu.VMEM((sb,), jnp.int32),
            pltpu.VMEM((sb,), jnp.int32),
            pltpu.VMEM((sb,), jnp.int32),
            pltpu.VMEM((sb,), jnp.float32),
            pltpu.VMEM((nch, c), jnp.int32),
            pltpu.VMEM((nch, c), jnp.int32),
            pltpu.VMEM((c, d), jnp.float32),
            pltpu.VMEM((c, d), jnp.float32),
            pltpu.VMEM((c, d), jnp.float32),
            pltpu.VMEM_SHARED((n, d), jnp.float32),
            pltpu.SemaphoreType.DMA,
            pltpu.SemaphoreType.DMA,
            pltpu.SemaphoreType.DMA,
            pltpu.SemaphoreType.DMA,
            pltpu.SemaphoreType.DMA,
            pltpu.SemaphoreType.DMA,
        ],
    )
    def agg_kernel(xt_hbm, ei_hbm, type_hbm, norm_hbm, out_hbm,
                   srcv, typev, dstv, normv, gidx2, didx2,
                   rows0, rows1, rows2, acc,
                   gsem0, gsem1, gsem2, ssem0, ssem1, ssem2):
        bufs = (rows0, rows1, rows2)
        gsems = (gsem0, gsem1, gsem2)
        ssems = (ssem0, ssem1, ssem2)
        cid = lax.axis_index("c")
        sid = lax.axis_index("s")
        wid = cid * NS + sid
        base = wid * epw

        zero = jnp.zeros((L,), jnp.float32)
        def zrows(i, carry):
            for j in range(d // L):
                rows0[i, pl.ds(j * L, L)] = zero
            return carry
        lax.fori_loop(0, c, zrows, 0)

        def zacc(k, carry):
            rc = sid + k * NS
            @pl.when(rc < nrc)
            def _():
                pltpu.sync_copy(rows0, acc.at[pl.ds(rc * c, c)])
            return carry
        lax.fori_loop(0, kmax, zacc, 0)
        plsc.subcore_barrier()

        dnums = lax.GatherDimensionNumbers(
            offset_dims=(), collapsed_slice_dims=(0,), start_index_map=(0,))

        def start_gather(i, b):
            @pl.when(i < nch)
            def _():
                pltpu.async_copy(xt_hbm.at[gidx2.at[i]], bufs[b], gsems[b])

        def wait_gather(b):
            pltpu.make_async_copy(
                xt_hbm.at[gidx2.at[0]], bufs[b], gsems[b]).wait()

        def start_scatter(i, b):
            pltpu.async_copy(bufs[b], acc.at[didx2.at[i]], ssems[b], add=True)

        def wait_scatter(b):
            pltpu.make_async_copy(
                bufs[b], acc.at[didx2.at[0]], ssems[b]).wait()

        def scale(i, b):
            buf = bufs[b]
            def group(g, inner2):
                nv = normv[pl.ds(i * c + g * L, L)]
                for jl in range(L):
                    lane = jnp.full((L, 1), jl, jnp.int32)
                    bc = lax.gather(
                        nv, lane, dnums, (1,),
                        mode=lax.GatherScatterMode.PROMISE_IN_BOUNDS)
                    row = g * L + jl
                    for k in range(d // L):
                        sl = pl.ds(k * L, L)
                        buf[row, sl] = buf[row, sl] * bc
                return inner2
            lax.fori_loop(0, nv_groups, group, 0)

        ntrip = (nch + 2) // 3

        def super_chunk(si, carry):
            sbase = base + si * sb
            pltpu.sync_copy(ei_hbm.at[pl.ds(sbase, sb)], srcv)
            pltpu.sync_copy(type_hbm.at[pl.ds(sbase, sb)], typev)
            pltpu.sync_copy(ei_hbm.at[pl.ds(e + sbase, sb)], dstv)
            pltpu.sync_copy(norm_hbm.at[pl.ds(sbase, sb)], normv)

            def prep(i, inner):
                for j in range(nv_groups):
                    off = i * c + j * L
                    s = srcv[pl.ds(off, L)]
                    t = typev[pl.ds(off, L)]
                    gidx2[i, pl.ds(j * L, L)] = t * n + s
                    didx2[i, pl.ds(j * L, L)] = dstv[pl.ds(off, L)]
                return inner
            lax.fori_loop(0, nch, prep, 0)

            start_gather(0, 0)
            start_gather(1, 1)

            def triple(j, inner):
                i = 3 * j
                wait_gather(0)
                scale(i, 0)
                start_scatter(i, 0)
                @pl.when(i > 0)
                def _():
                    wait_scatter(2)
                start_gather(i + 2, 2)
                @pl.when(i + 1 < nch)
                def _():
                    wait_gather(1)
                    scale(i + 1, 1)
                    start_scatter(i + 1, 1)
                @pl.when(i + 3 < nch)
                def _():
                    wait_scatter(0)
                    start_gather(i + 3, 0)
                @pl.when(i + 2 < nch)
                def _():
                    wait_gather(2)
                    scale(i + 2, 2)
                    start_scatter(i + 2, 2)
                @pl.when(i + 4 < nch)
                def _():
                    wait_scatter(1)
                    start_gather(i + 4, 1)
                return inner
            lax.fori_loop(0, ntrip, triple, 0)

            waited = set()
            for j in range(ntrip):
                if j > 0:
                    waited.add(3 * j - 1)
                if 3 * j + 3 < nch:
                    waited.add(3 * j)
                if 3 * j + 4 < nch and 3 * j + 1 < nch:
                    waited.add(3 * j + 1)
            for k in range(nch):
                if k not in waited:
                    wait_scatter(k % 3)
            return carry
        lax.fori_loop(0, nsuper, super_chunk, 0)
        plsc.subcore_barrier()

        def dump(k, carry):
            rc = sid + k * NS
            @pl.when(rc < nrc)
            def _():
                sl = pl.ds(rc * c, c)
                pltpu.sync_copy(acc.at[sl], rows0)
                pltpu.sync_copy(rows0, out_hbm.at[cid, sl])
            return carry
        lax.fori_loop(0, kmax, dump, 0)

    return agg_kernel(xt, ei_flat, etype, enorm)


def kernel(x, edge_index, edge_type, edge_norm, W_rel, W_root, b):
    n, d = x.shape
    xt = _transform(x, W_rel)
    partials = _sc_aggregate(xt, edge_index.reshape(-1), edge_type,
                             edge_norm, n)
    return _combine(partials, x, W_root, b.reshape(1, d))
